# baseline (speedup 1.0000x reference)
import dataclasses
import functools

import jax
import jax.numpy as jnp
from jax import lax
from jax.experimental import pallas as pl
from jax.experimental.pallas import tpu as pltpu
from jax.experimental.pallas import tpu_sc as plsc

NUM_CORES = 2
NUM_SUBCORES = 16
NUM_WORKERS = NUM_CORES * NUM_SUBCORES
LANES = 16
PACK = 4
CHUNK = 128
BLK = 8192


def _pack_table(t_t, n_rows, dim):
    def body(t_ref, out_ref):
        out_ref[...] = t_ref[...].T

    transposed = pl.pallas_call(
        body,
        grid=(pl.cdiv(n_rows, BLK),),
        in_specs=[pl.BlockSpec((dim, BLK), lambda i: (0, i))],
        out_specs=pl.BlockSpec((BLK, dim), lambda i: (i, 0)),
        out_shape=jax.ShapeDtypeStruct((n_rows, dim), jnp.float32),
        compiler_params=pltpu.CompilerParams(
            dimension_semantics=("parallel",)),
    )(t_t)
    return transposed.reshape(n_rows // PACK, PACK * dim)


def kernel(x, table):
    (batch,) = x.shape
    n_rows, dim = table.shape
    packed = _pack_table(table.T, n_rows, dim)
    b_per_w = batch // NUM_WORKERS
    n_chunks = b_per_w // CHUNK
    x2 = x.astype(jnp.int32).reshape(batch // CHUNK, CHUNK)
    mesh = plsc.VectorSubcoreMesh(core_axis_name="c", subcore_axis_name="s")
    cp = pltpu.CompilerParams()
    if "needs_layout_passes" in pltpu.CompilerParams.__dataclass_fields__:
        cp = dataclasses.replace(cp, needs_layout_passes=False)

    @functools.partial(
        pl.kernel,
        mesh=mesh,
        compiler_params=cp,
        out_type=jax.ShapeDtypeStruct((batch, dim), table.dtype),
        scratch_types=[
            pltpu.VMEM((n_chunks, CHUNK), jnp.int32),
            pltpu.VMEM((n_chunks, CHUNK), jnp.int32),
            pltpu.VMEM((n_chunks, CHUNK), jnp.int32),
            pltpu.VMEM((CHUNK, PACK * dim), jnp.float32),
            pltpu.VMEM((b_per_w, dim), jnp.float32),
            pltpu.SemaphoreType.DMA,
        ],
    )
    def gather_kernel(tab_hbm, idx_hbm, out_hbm, idx_v, row_v, col_v,
                      rows_v, out_v, sem):
        wid = lax.axis_index("s") * NUM_CORES + lax.axis_index("c")
        base = wid * b_per_w
        pltpu.sync_copy(idx_hbm.at[pl.ds(wid * n_chunks, n_chunks)], idx_v)

        @pl.loop(0, n_chunks)
        def _(k):
            @pl.loop(0, CHUNK, step=LANES)
            def _(c):
                v = idx_v[k, pl.ds(c, LANES)]
                row_v[k, pl.ds(c, LANES)] = lax.shift_right_logical(v, 2)
                col_v[k, pl.ds(c, LANES)] = lax.shift_left(
                    lax.bitwise_and(v, 3), 5)

        @pl.loop(0, n_chunks)
        def _(k):
            pltpu.async_copy(tab_hbm.at[row_v.at[k]], rows_v, sem).wait()

            @pl.loop(0, CHUNK, step=LANES)
            def _(j0):
                riota = lax.iota(jnp.int32, LANES) + j0
                colb = col_v[k, pl.ds(j0, LANES)]
                rout = riota + k * CHUNK
                for c in range(dim):
                    val = plsc.load_gather(rows_v, [riota, colb + c])
                    plsc.store_scatter(out_v, [rout, jnp.full((LANES,), c,
                                                              jnp.int32)], val)

        pltpu.sync_copy(out_v, out_hbm.at[pl.ds(base, b_per_w)])

    return gather_kernel(packed, x2)


# speedup vs baseline: 1.9865x; 1.9865x over previous
import dataclasses
import functools

import jax
import jax.numpy as jnp
from jax import lax
from jax.experimental import pallas as pl
from jax.experimental.pallas import tpu as pltpu
from jax.experimental.pallas import tpu_sc as plsc

NUM_CORES = 2
NUM_SUBCORES = 16
NUM_WORKERS = NUM_CORES * NUM_SUBCORES
LANES = 16
PACK = 8
CHUNK = 128
BLK = 8192


def _pack_table(t_t, n_rows, dim):
    half = dim // 2

    def body(t_ref, out_ref):
        b = t_ref[...].astype(jnp.bfloat16)
        pairs = pltpu.bitcast(b, jnp.float32)
        t = pairs.T
        t3 = t.reshape(BLK // PACK, PACK, half)
        for q in range(PACK):
            out_ref[:, half * q:half * (q + 1)] = t3[:, q, :]

    return pl.pallas_call(
        body,
        grid=(pl.cdiv(n_rows, BLK),),
        in_specs=[pl.BlockSpec((dim, BLK), lambda i: (0, i))],
        out_specs=pl.BlockSpec((BLK // PACK, PACK * half), lambda i: (i, 0)),
        out_shape=jax.ShapeDtypeStruct((n_rows // PACK, PACK * half),
                                       jnp.float32),
        compiler_params=pltpu.CompilerParams(
            dimension_semantics=("parallel",)),
    )(t_t)


def kernel(x, table):
    (batch,) = x.shape
    n_rows, dim = table.shape
    half = dim // 2
    packed = _pack_table(table.T, n_rows, dim)
    b_per_w = batch // NUM_WORKERS
    n_chunks = b_per_w // CHUNK
    x2 = x.astype(jnp.int32).reshape(batch // CHUNK, CHUNK)
    mesh = plsc.VectorSubcoreMesh(core_axis_name="c", subcore_axis_name="s")
    cp = pltpu.CompilerParams()
    if "needs_layout_passes" in pltpu.CompilerParams.__dataclass_fields__:
        cp = dataclasses.replace(cp, needs_layout_passes=False)

    @functools.partial(
        pl.kernel,
        mesh=mesh,
        compiler_params=cp,
        out_type=jax.ShapeDtypeStruct((batch, half), jnp.float32),
        scratch_types=[
            pltpu.VMEM((n_chunks, CHUNK), jnp.int32),
            pltpu.VMEM((n_chunks, CHUNK), jnp.int32),
            pltpu.VMEM((n_chunks, CHUNK), jnp.int32),
            pltpu.VMEM((CHUNK, PACK * half), jnp.float32),
            pltpu.VMEM((b_per_w, half), jnp.float32),
            pltpu.SemaphoreType.DMA,
        ],
    )
    def gather_kernel(tab_hbm, idx_hbm, out_hbm, idx_v, row_v, col_v,
                      rows_v, out_v, sem):
        wid = lax.axis_index("s") * NUM_CORES + lax.axis_index("c")
        base = wid * b_per_w
        pltpu.sync_copy(idx_hbm.at[pl.ds(wid * n_chunks, n_chunks)], idx_v)

        @pl.loop(0, n_chunks)
        def _(k):
            @pl.loop(0, CHUNK, step=LANES)
            def _(c):
                v = idx_v[k, pl.ds(c, LANES)]
                row_v[k, pl.ds(c, LANES)] = lax.shift_right_logical(v, 3)
                col_v[k, pl.ds(c, LANES)] = lax.shift_left(
                    lax.bitwise_and(v, PACK - 1), 4)

        @pl.loop(0, n_chunks)
        def _(k):
            pltpu.async_copy(tab_hbm.at[row_v.at[k]], rows_v, sem).wait()

            @pl.loop(0, CHUNK, step=LANES)
            def _(j0):
                riota = lax.iota(jnp.int32, LANES) + j0
                colb = col_v[k, pl.ds(j0, LANES)]
                rout = riota + k * CHUNK
                for c in range(half):
                    val = plsc.load_gather(rows_v, [riota, colb + c])
                    plsc.store_scatter(out_v, [rout, jnp.full((LANES,), c,
                                                              jnp.int32)], val)

        pltpu.sync_copy(out_v, out_hbm.at[pl.ds(base, b_per_w)])

    out_pairs = gather_kernel(packed, x2)
    out_bf16 = lax.bitcast_convert_type(out_pairs, jnp.bfloat16)
    return out_bf16.reshape(batch, dim).astype(jnp.float32)


# speedup vs baseline: 2.0289x; 1.0214x over previous
import dataclasses
import functools

import jax
import jax.numpy as jnp
from jax import lax
from jax.experimental import pallas as pl
from jax.experimental.pallas import tpu as pltpu
from jax.experimental.pallas import tpu_sc as plsc

NUM_CORES = 2
NUM_SUBCORES = 16
NUM_WORKERS = NUM_CORES * NUM_SUBCORES
LANES = 16
PACK = 8
CHUNK = 128
BLK = 32768


def _pack_table(t_t, n_rows, dim):
    half = dim // 2

    def body(t_ref, out_ref):
        b = t_ref[...].astype(jnp.bfloat16)
        pairs = pltpu.bitcast(b, jnp.float32)
        t = pairs.T
        t3 = t.reshape(BLK // PACK, PACK, half)
        for q in range(PACK):
            out_ref[:, half * q:half * (q + 1)] = t3[:, q, :]

    return pl.pallas_call(
        body,
        grid=(pl.cdiv(n_rows, BLK),),
        in_specs=[pl.BlockSpec((dim, BLK), lambda i: (0, i))],
        out_specs=pl.BlockSpec((BLK // PACK, PACK * half), lambda i: (i, 0)),
        out_shape=jax.ShapeDtypeStruct((n_rows // PACK, PACK * half),
                                       jnp.float32),
        compiler_params=pltpu.CompilerParams(
            dimension_semantics=("parallel",)),
    )(t_t)


def kernel(x, table):
    (batch,) = x.shape
    n_rows, dim = table.shape
    half = dim // 2
    packed = _pack_table(table.T, n_rows, dim)
    b_per_w = batch // NUM_WORKERS
    n_chunks = b_per_w // CHUNK
    x2 = x.astype(jnp.int32).reshape(batch // CHUNK, CHUNK)
    mesh = plsc.VectorSubcoreMesh(core_axis_name="c", subcore_axis_name="s")
    cp = pltpu.CompilerParams()
    if "needs_layout_passes" in pltpu.CompilerParams.__dataclass_fields__:
        cp = dataclasses.replace(cp, needs_layout_passes=False)

    @functools.partial(
        pl.kernel,
        mesh=mesh,
        compiler_params=cp,
        out_type=jax.ShapeDtypeStruct((batch, half), jnp.float32),
        scratch_types=[
            pltpu.VMEM((n_chunks, CHUNK), jnp.int32),
            pltpu.VMEM((n_chunks, CHUNK), jnp.int32),
            pltpu.VMEM((n_chunks, CHUNK), jnp.int32),
            pltpu.VMEM((CHUNK, PACK * half), jnp.float32),
            pltpu.VMEM((b_per_w, half), jnp.float32),
            pltpu.SemaphoreType.DMA,
        ],
    )
    def gather_kernel(tab_hbm, idx_hbm, out_hbm, idx_v, row_v, col_v,
                      rows_v, out_v, sem):
        wid = lax.axis_index("s") * NUM_CORES + lax.axis_index("c")
        base = wid * b_per_w
        pltpu.sync_copy(idx_hbm.at[pl.ds(wid * n_chunks, n_chunks)], idx_v)

        @pl.loop(0, n_chunks)
        def _(k):
            @pl.loop(0, CHUNK, step=LANES)
            def _(c):
                v = idx_v[k, pl.ds(c, LANES)]
                row_v[k, pl.ds(c, LANES)] = lax.shift_right_logical(v, 3)
                col_v[k, pl.ds(c, LANES)] = lax.shift_left(
                    lax.bitwise_and(v, PACK - 1), 4)

        @pl.loop(0, n_chunks)
        def _(k):
            pltpu.async_copy(tab_hbm.at[row_v.at[k]], rows_v, sem).wait()

            @pl.loop(0, CHUNK, step=LANES)
            def _(j0):
                riota = lax.iota(jnp.int32, LANES) + j0
                colb = col_v[k, pl.ds(j0, LANES)]
                rout = riota + k * CHUNK
                for c in range(half):
                    val = plsc.load_gather(rows_v, [riota, colb + c])
                    plsc.store_scatter(out_v, [rout, jnp.full((LANES,), c,
                                                              jnp.int32)], val)

        pltpu.sync_copy(out_v, out_hbm.at[pl.ds(base, b_per_w)])

    out_pairs = gather_kernel(packed, x2)
    out_bf16 = lax.bitcast_convert_type(out_pairs, jnp.bfloat16)
    return out_bf16.reshape(batch, dim).astype(jnp.float32)


# speedup vs baseline: 2.0543x; 1.0125x over previous
import dataclasses
import functools

import jax
import jax.numpy as jnp
from jax import lax
from jax.experimental import pallas as pl
from jax.experimental.pallas import tpu as pltpu
from jax.experimental.pallas import tpu_sc as plsc

NUM_CORES = 2
NUM_SUBCORES = 16
NUM_WORKERS = NUM_CORES * NUM_SUBCORES
LANES = 16
PACK = 8
CHUNK = 128
BLK = 32768


def _pack_table(t_t, n_rows, dim):
    half = dim // 2

    def body(t_ref, out_ref):
        b = t_ref[...].astype(jnp.bfloat16)
        pairs = pltpu.bitcast(b, jnp.float32)
        t = pairs.T
        t3 = t.reshape(BLK // PACK, PACK, half)
        for q in range(PACK):
            out_ref[:, half * q:half * (q + 1)] = t3[:, q, :]

    return pl.pallas_call(
        body,
        grid=(pl.cdiv(n_rows, BLK),),
        in_specs=[pl.BlockSpec((dim, BLK), lambda i: (0, i))],
        out_specs=pl.BlockSpec((BLK // PACK, PACK * half), lambda i: (i, 0)),
        out_shape=jax.ShapeDtypeStruct((n_rows // PACK, PACK * half),
                                       jnp.float32),
        compiler_params=pltpu.CompilerParams(
            dimension_semantics=("parallel",)),
    )(t_t)


def kernel(x, table):
    (batch,) = x.shape
    n_rows, dim = table.shape
    half = dim // 2
    packed = _pack_table(table.T, n_rows, dim)
    b_per_w = batch // NUM_WORKERS
    n_chunks = b_per_w // CHUNK
    x2 = x.astype(jnp.int32).reshape(batch // CHUNK, CHUNK)
    mesh = plsc.VectorSubcoreMesh(core_axis_name="c", subcore_axis_name="s")
    cp = pltpu.CompilerParams()
    if "needs_layout_passes" in pltpu.CompilerParams.__dataclass_fields__:
        cp = dataclasses.replace(cp, needs_layout_passes=False)

    @functools.partial(
        pl.kernel,
        mesh=mesh,
        compiler_params=cp,
        out_type=jax.ShapeDtypeStruct((batch, half), jnp.float32),
        scratch_types=[
            pltpu.VMEM((n_chunks, CHUNK), jnp.int32),
            pltpu.VMEM((n_chunks, CHUNK), jnp.int32),
            pltpu.VMEM((n_chunks, CHUNK), jnp.int32),
            pltpu.VMEM((2, CHUNK, PACK * half), jnp.float32),
            pltpu.VMEM((b_per_w, half), jnp.float32),
            pltpu.SemaphoreType.DMA,
        ],
    )
    def gather_kernel(tab_hbm, idx_hbm, out_hbm, idx_v, row_v, col_v,
                      rows_v, out_v, sem):
        wid = lax.axis_index("s") * NUM_CORES + lax.axis_index("c")
        base = wid * b_per_w
        pltpu.sync_copy(idx_hbm.at[pl.ds(wid * n_chunks, n_chunks)], idx_v)

        @pl.loop(0, n_chunks)
        def _(k):
            @pl.loop(0, CHUNK, step=LANES)
            def _(c):
                v = idx_v[k, pl.ds(c, LANES)]
                row_v[k, pl.ds(c, LANES)] = lax.shift_right_logical(v, 3)
                col_v[k, pl.ds(c, LANES)] = lax.shift_left(
                    lax.bitwise_and(v, PACK - 1), 4)

        copies = [
            pltpu.async_copy(tab_hbm.at[row_v.at[k]], rows_v.at[k], sem)
            for k in range(2)
        ]
        for k in range(n_chunks):
            copies[k % 2].wait()
            chunk_rows = rows_v.at[k % 2]

            @pl.loop(0, CHUNK, step=LANES)
            def _(j0, k=k, chunk_rows=chunk_rows):
                riota = lax.iota(jnp.int32, LANES) + j0
                colb = col_v[k, pl.ds(j0, LANES)]
                rout = riota + k * CHUNK
                for c in range(half):
                    val = plsc.load_gather(chunk_rows, [riota, colb + c])
                    plsc.store_scatter(out_v, [rout, jnp.full((LANES,), c,
                                                              jnp.int32)], val)

            if k + 2 < n_chunks:
                copies[k % 2] = pltpu.async_copy(
                    tab_hbm.at[row_v.at[k + 2]], rows_v.at[k % 2], sem)

        pltpu.sync_copy(out_v, out_hbm.at[pl.ds(base, b_per_w)])

    out_pairs = gather_kernel(packed, x2)
    out_bf16 = lax.bitcast_convert_type(out_pairs, jnp.bfloat16)
    return out_bf16.reshape(batch, dim).astype(jnp.float32)
